# baseline (speedup 1.0000x reference)
import functools

import jax
import jax.numpy as jnp
import numpy as np
from jax import lax
from jax.experimental import pallas as pl
from jax.experimental.pallas import tpu as pltpu
from jax.experimental.pallas import tpu_sc as plsc

B = 16384
NUM = 13
CAT = 26
V = 100000
D = 64
T = 1 + NUM + CAT

NC = 2
NS = 16
NW = NC * NS
BPW = B // NW
NB = 16
NCHUNK = BPW // NB
IDXC = NB * CAT
GSUB = 4
GIDX = IDXC // GSUB
DC = D // 16


def _body(xnum_hbm, xcat_hbm, numw_hbm, numb_hbm, emb_hbm, cls_hbm, offs_hbm,
          out_hbm, offs_v, numw_v, numb_v, cls_v, idx_v, numv, catbuf, numbuf,
          sem):
    cid = lax.axis_index("c")
    sid = lax.axis_index("s")
    wid = sid * NC + cid
    base = wid * BPW

    pltpu.sync_copy(offs_hbm, offs_v)
    pltpu.sync_copy(numw_hbm, numw_v)
    pltpu.sync_copy(numb_hbm, numb_v)
    pltpu.sync_copy(cls_hbm, cls_v)

    def chunk(g, carry):
        b0 = base + g * NB

        pltpu.sync_copy(xcat_hbm.at[pl.ds(b0 * CAT, IDXC)], idx_v)
        for k in range(IDXC // 16):
            s = pl.ds(k * 16, 16)
            idx_v[s] = idx_v[s] + offs_v[s]

        descs = []
        for sub in range(GSUB):
            s = pl.ds(sub * GIDX, GIDX)
            descs.append(
                pltpu.async_copy(emb_hbm.at[idx_v.at[s]], catbuf.at[s], sem))

        pltpu.sync_copy(xnum_hbm.at[pl.ds(b0 * NUM, NB * NUM)], numv)

        def cls_row(b, c):
            for dc in range(DC):
                s = pl.ds(dc * 16, 16)
                numbuf[b, 0, s] = cls_v[s]
            return c

        lax.fori_loop(0, NB, cls_row, 0)

        zeros = jnp.zeros((16,), jnp.float32)
        for i in range(NUM):
            wch = [numw_v[i, pl.ds(dc * 16, 16)] for dc in range(DC)]
            bch = [numb_v[i, pl.ds(dc * 16, 16)] for dc in range(DC)]

            def num_row(b, c, i=i, wch=wch, bch=bch):
                pos = jnp.full((16,), b * NUM + i, jnp.int32)
                vb = plsc.load_gather(numv, [pos])
                nan = vb != vb
                for dc in range(DC):
                    tok = jnp.where(nan, zeros, vb * wch[dc] + bch[dc])
                    numbuf[b, i + 1, pl.ds(dc * 16, 16)] = tok
                return c

            lax.fori_loop(0, NB, num_row, 0)

        for d in descs:
            d.wait()

        pltpu.sync_copy(numbuf,
                        out_hbm.at[pl.ds(b0, NB), pl.ds(0, 1 + NUM), :])
        def cat_out(b, c):
            pltpu.sync_copy(catbuf.at[pl.ds(b * CAT, CAT)],
                            out_hbm.at[b0 + b, pl.ds(1 + NUM, CAT), :])
            return c

        lax.fori_loop(0, NB, cat_out, 0)
        return carry

    lax.fori_loop(0, NCHUNK, chunk, 0)


_sc_call = functools.partial(
    pl.kernel,
    out_type=jax.ShapeDtypeStruct((B, T, D), jnp.float32),
    mesh=plsc.VectorSubcoreMesh(
        core_axis_name="c", subcore_axis_name="s", num_cores=NC,
        num_subcores=NS),
    compiler_params=pltpu.CompilerParams(
        needs_layout_passes=False, use_tc_tiling_on_sc=False),
    scratch_types=[
        pltpu.VMEM((IDXC,), jnp.int32),
        pltpu.VMEM((NUM, D), jnp.float32),
        pltpu.VMEM((NUM, D), jnp.float32),
        pltpu.VMEM((D,), jnp.float32),
        pltpu.VMEM((IDXC,), jnp.int32),
        pltpu.VMEM((NB * NUM,), jnp.float32),
        pltpu.VMEM((IDXC, D), jnp.float32),
        pltpu.VMEM((NB, 1 + NUM, D), jnp.float32),
        pltpu.SemaphoreType.DMA,
    ],
)


@jax.jit
def _tokenize(x_num_flat, x_cat_flat, num_w, num_b, emb_flat, cls_flat, offs):
    return _sc_call(_body)(x_num_flat, x_cat_flat, num_w, num_b, emb_flat,
                           cls_flat, offs)


def kernel(x_num, x_cat, num_w, num_b, emb, cls_token):
    x_num_flat = x_num.reshape(B * NUM)
    x_cat_flat = x_cat.reshape(B * CAT).astype(jnp.int32)
    emb_flat = emb.reshape(CAT * V, D)
    cls_flat = cls_token.reshape(D)
    offs = jnp.asarray(np.tile(np.arange(CAT, dtype=np.int32) * V, NB))
    return _tokenize(x_num_flat, x_cat_flat, num_w, num_b, emb_flat, cls_flat,
                     offs)
